# baseline (speedup 1.0000x reference)
import jax
import jax.numpy as jnp
from jax.experimental import pallas as pl

_MAP_INDEX = 50000
_LANES = 128
_BLK_COL = _MAP_INDEX // _LANES
_COL_IN_BLK = _MAP_INDEX % _LANES


def _gather_kernel(x_ref, o_ref):
    col = x_ref[:, _COL_IN_BLK:_COL_IN_BLK + 1]
    o_ref[:, 0:1] = 1.0 - col
    o_ref[:, 1:2] = col


def kernel(logits):
    b, _ = logits.shape
    return pl.pallas_call(
        _gather_kernel,
        grid=(1,),
        in_specs=[pl.BlockSpec((b, _LANES), lambda i: (0, _BLK_COL))],
        out_specs=pl.BlockSpec((b, 2), lambda i: (0, 0)),
        out_shape=jax.ShapeDtypeStruct((b, 2), logits.dtype),
    )(logits)
